# baseline (speedup 1.0000x reference)
import functools

import jax
import jax.numpy as jnp
from jax import lax
from jax.experimental import pallas as pl
from jax.experimental.pallas import tpu as pltpu
from jax.experimental.pallas import tpu_sc as plsc

EMB_DIM = 32

_info = plsc.get_sparse_core_info()
_NC, _NS = _info.num_cores, _info.num_subcores
_NW = _NC * _NS

_CHUNK = 256
_K = 13


_DBLK = 128
_DK = 4


@functools.cache
def _make_detile(V: int, D: int):
    nblk = V // _DBLK
    tail = V - nblk * _DBLK
    base_cnt = nblk // _NW
    extra = nblk - base_cnt * _NW

    mesh = plsc.VectorSubcoreMesh(core_axis_name="c", subcore_axis_name="s")

    @functools.partial(
        pl.kernel,
        mesh=mesh,
        out_type=jax.ShapeDtypeStruct((V * D,), jnp.float32),
        scratch_types=(
            [pltpu.VMEM((_DBLK, D), jnp.float32) for _ in range(_DK)]
            + [pltpu.VMEM((_DBLK * D,), jnp.float32) for _ in range(_DK)]
            + [pltpu.SemaphoreType.DMA for _ in range(_DK)]
            + [pltpu.SemaphoreType.DMA for _ in range(_DK)]
        ),
        compiler_params=pltpu.CompilerParams(
            use_tc_tiling_on_sc=True, needs_layout_passes=False),
    )
    def detile_kernel(tab_hbm, out_hbm, *bufs):
        src_v = bufs[:_DK]
        stg_v = bufs[_DK:2 * _DK]
        rsem = bufs[2 * _DK:3 * _DK]
        wsem = bufs[3 * _DK:4 * _DK]
        wid = lax.axis_index("s") * _NC + lax.axis_index("c")

        def bridge(src, stg, nr):
            for r in range(nr):
                for h in range(D // 16):
                    stg[pl.ds(r * D + h * 16, 16)] = src[r, pl.ds(h * 16, 16)]

        def do_block(blk, b):
            v0 = blk * _DBLK
            pltpu.async_copy(
                tab_hbm.at[pl.ds(v0, _DBLK)], src_v[b], rsem[b]).wait()
            bridge(src_v[b], stg_v[b], _DBLK)
            return pltpu.async_copy(
                stg_v[b], out_hbm.at[pl.ds(v0 * D, _DBLK * D)], wsem[b])

        ngroup = base_cnt // _DK
        assert base_cnt % _DK == 0

        def group(g, carry):
            wcopies = []
            for b in range(_DK):
                j = g * _DK + b
                wcopies.append(do_block(wid + j * _NW, b))
            for w in wcopies:
                w.wait()
            return carry

        lax.fori_loop(0, ngroup, group, 0)

        @pl.when(wid < extra)
        def _():
            do_block(base_cnt * _NW + wid, 0).wait()

        if tail:
            @pl.when(wid == _NW - 1)
            def _():
                v0 = nblk * _DBLK
                pltpu.async_copy(
                    tab_hbm.at[pl.ds(v0, tail)],
                    src_v[1].at[pl.ds(0, tail)], rsem[1]).wait()
                bridge(src_v[1], stg_v[1], tail)
                pltpu.async_copy(
                    stg_v[1].at[pl.ds(0, tail * D)],
                    out_hbm.at[pl.ds(v0 * D, tail * D)], wsem[1]).wait()

    return detile_kernel


@functools.cache
def _make_gather(B: int):
    b_per_w = B // _NW
    nchunk = b_per_w // _CHUNK
    ngroup = nchunk // _K
    assert B % _NW == 0 and b_per_w % _CHUNK == 0 and nchunk % _K == 0

    mesh = plsc.VectorSubcoreMesh(core_axis_name="c", subcore_axis_name="s")

    @functools.partial(
        pl.kernel,
        mesh=mesh,
        out_type=jax.ShapeDtypeStruct((B, EMB_DIM), jnp.float32),
        scratch_types=[
            pltpu.VMEM((nchunk, _CHUNK), jnp.int32),
            pltpu.VMEM((_K, _CHUNK, EMB_DIM), jnp.float32),
            pltpu.SemaphoreType.DMA((_K,)),
        ],
        compiler_params=pltpu.CompilerParams(use_tc_tiling_on_sc=False),
    )
    def gather_kernel(keys_hbm, table_hbm, out_hbm, idx_v, rows_v, gsem):
        wid = lax.axis_index("s") * _NC + lax.axis_index("c")
        base = wid * b_per_w
        pltpu.sync_copy(keys_hbm.at[wid], idx_v)

        def group(g, carry):
            copies = []
            for b in range(_K):
                c = g * _K + b
                copies.append(
                    pltpu.async_copy(
                        table_hbm.at[idx_v.at[c]], rows_v.at[b], gsem.at[b]
                    )
                )
            for b in range(_K):
                c = g * _K + b
                copies[b].wait()
                pltpu.sync_copy(
                    rows_v.at[b],
                    out_hbm.at[pl.ds(base + c * _CHUNK, _CHUNK)],
                )
            return carry

        lax.fori_loop(0, ngroup, group, 0)

    return gather_kernel


def kernel(keys, table):
    batch, fields = keys.shape
    B = batch * fields
    b_per_w = B // _NW
    nchunk = b_per_w // _CHUNK
    karr = keys.reshape(-1).astype(jnp.int32).reshape(_NW, nchunk, _CHUNK)
    V, D = table.shape
    tab_flat = _make_detile(V, D)(table)
    out = _make_gather(B)(karr, tab_flat.reshape(V, D))
    return out.reshape(batch, fields, EMB_DIM)


# speedup vs baseline: 1.2534x; 1.2534x over previous
import functools

import jax
import jax.numpy as jnp
from jax import lax
from jax.experimental import pallas as pl
from jax.experimental.pallas import tpu as pltpu
from jax.experimental.pallas import tpu_sc as plsc

EMB_DIM = 32

_info = plsc.get_sparse_core_info()
_NC, _NS = _info.num_cores, _info.num_subcores
_NW = _NC * _NS

_CHUNK = 256
_K = 13


@functools.cache
def _make_gather(B: int):
    b_per_w = B // _NW
    nchunk = b_per_w // _CHUNK
    ngroup = nchunk // _K
    assert B % _NW == 0 and b_per_w % _CHUNK == 0 and nchunk % _K == 0

    mesh = plsc.VectorSubcoreMesh(core_axis_name="c", subcore_axis_name="s")

    @functools.partial(
        pl.kernel,
        mesh=mesh,
        out_type=jax.ShapeDtypeStruct((B, EMB_DIM), jnp.float32),
        scratch_types=[
            pltpu.VMEM((nchunk, _CHUNK), jnp.int32),
            pltpu.VMEM((_K, _CHUNK, EMB_DIM), jnp.float32),
            pltpu.SemaphoreType.DMA((_K,)),
        ],
        compiler_params=pltpu.CompilerParams(use_tc_tiling_on_sc=False),
    )
    def gather_kernel(keys_hbm, table_hbm, out_hbm, idx_v, rows_v, gsem):
        wid = lax.axis_index("s") * _NC + lax.axis_index("c")
        base = wid * b_per_w
        pltpu.sync_copy(keys_hbm.at[wid], idx_v)

        def group(g, carry):
            copies = []
            for b in range(_K):
                c = g * _K + b
                copies.append(
                    pltpu.async_copy(
                        table_hbm.at[idx_v.at[c]], rows_v.at[b], gsem.at[b]
                    )
                )
            for b in range(_K):
                c = g * _K + b
                copies[b].wait()
                pltpu.sync_copy(
                    rows_v.at[b],
                    out_hbm.at[pl.ds(base + c * _CHUNK, _CHUNK)],
                )
            return carry

        lax.fori_loop(0, ngroup, group, 0)

    return gather_kernel


def kernel(keys, table):
    batch, fields = keys.shape
    B = batch * fields
    b_per_w = B // _NW
    nchunk = b_per_w // _CHUNK
    karr = keys.reshape(-1).astype(jnp.int32).reshape(_NW, nchunk, _CHUNK)
    out = _make_gather(B)(karr, table)
    return out.reshape(batch, fields, EMB_DIM)
